# baseline (speedup 1.0000x reference)
import dataclasses
import functools

import jax
import jax.numpy as jnp
from jax import lax
from jax.experimental import pallas as pl
from jax.experimental.pallas import tpu as pltpu
from jax.experimental.pallas import tpu_sc as plsc

VOCAB = 1000
EMB = 128
FIELDS = 26
SUB = 8
NW = 32

G = 2
WIN = SUB * FIELDS
NHALF = 2


def kernel(indices, tables):
    batch = indices.shape[0]
    ngrp = batch // SUB
    grp_w = ngrp // NW
    n = batch * FIELDS
    nw_ = n // NW
    nh = nw_ // NHALF
    gh = grp_w // NHALF
    flat_tables = tables.reshape(FIELDS * VOCAB, EMB)
    offs = (jnp.arange(FIELDS, dtype=indices.dtype) * VOCAB)[None, :]
    biased = jnp.pad(indices + offs, ((0, 0), (0, 128 - FIELDS))).reshape(batch * 128)

    mesh = plsc.VectorSubcoreMesh(core_axis_name="core", subcore_axis_name="subcore")

    cp = pltpu.CompilerParams()
    if "needs_layout_passes" in pltpu.CompilerParams.__dataclass_fields__:
        cp = dataclasses.replace(cp, needs_layout_passes=False)

    @functools.partial(
        pl.kernel,
        out_type=jax.ShapeDtypeStruct((ngrp, FIELDS, SUB, EMB), tables.dtype),
        mesh=mesh,
        scratch_types=[
            pltpu.VMEM((64 * 128,), jnp.int32),
            pltpu.VMEM((nw_,), jnp.int32),
            pltpu.SemaphoreType.DMA,
            pltpu.SemaphoreType.DMA,
        ],
        compiler_params=cp,
    )
    def gather_kernel(x_hbm, i_hbm, o_hbm, raw_v, idx_v, sem_a, sem_b):
        cid = lax.axis_index(("core", "subcore"))

        lane = lax.broadcasted_iota(jnp.int32, (16,), 0)
        bi16 = lane % SUB
        fh16 = lane // SUB

        @pl.loop(0, grp_w // 8)
        def _(c):
            pltpu.sync_copy(
                i_hbm.at[pl.ds((cid * grp_w + c * 8) * SUB * 128, 64 * 128)], raw_v
            )

            @pl.loop(0, 8)
            def _(g):
                @pl.loop(0, FIELDS // 2)
                def _(j):
                    src16 = (g * SUB + bi16) * 128 + 2 * j + fh16
                    vals = plsc.load_gather(raw_v, [src16])
                    idx_v[pl.ds((c * 8 + g) * WIN + j * 16, 16)] = vals

        def body(grid_idx, o_vmem):
            (i,) = grid_idx
            li = i - cid * (grp_w // G)
            o_flat = o_vmem.reshape(G * WIN, EMB)
            ca = pltpu.make_async_copy(
                x_hbm.at[idx_v.at[pl.ds(li * G * WIN, WIN)]],
                o_flat.at[pl.ds(0, WIN), :],
                sem_a,
            )
            cb = pltpu.make_async_copy(
                x_hbm.at[idx_v.at[pl.ds(li * G * WIN + WIN, WIN)]],
                o_flat.at[pl.ds(WIN, WIN), :],
                sem_b,
            )
            ca.start()
            cb.start()
            ca.wait()
            cb.wait()

        pltpu.emit_pipeline(
            body,
            grid=(ngrp // G,),
            out_specs=[
                pl.BlockSpec((G, FIELDS, SUB, EMB), index_map=lambda i: (i, 0, 0, 0))
            ],
            core_axis_name=("core", "subcore"),
            dimension_semantics=(pltpu.PARALLEL,),
            _explicit_indices=True,
        )(o_hbm)

    out4 = gather_kernel(flat_tables, biased)
    return out4.transpose(0, 2, 1, 3).reshape(batch, FIELDS * EMB)


# speedup vs baseline: 1.0397x; 1.0397x over previous
import dataclasses
import functools

import jax
import jax.numpy as jnp
from jax import lax
from jax.experimental import pallas as pl
from jax.experimental.pallas import tpu as pltpu
from jax.experimental.pallas import tpu_sc as plsc

VOCAB = 1000
EMB = 128
FIELDS = 26
SUB = 8
NW = 32

G = 2
WIN = SUB * FIELDS
RING = 4


def kernel(indices, tables):
    batch = indices.shape[0]
    ngrp = batch // SUB
    grp_w = ngrp // NW
    nsteps_w = grp_w // G
    n = batch * FIELDS
    nw_ = n // NW
    flat_tables = tables.reshape(FIELDS * VOCAB, EMB)
    offs = (jnp.arange(n, dtype=indices.dtype) % FIELDS) * VOCAB
    biased = indices.reshape(n) + offs

    mesh = plsc.VectorSubcoreMesh(core_axis_name="core", subcore_axis_name="subcore")

    cp = pltpu.CompilerParams()
    if "needs_layout_passes" in pltpu.CompilerParams.__dataclass_fields__:
        cp = dataclasses.replace(cp, needs_layout_passes=False)

    @functools.partial(
        pl.kernel,
        out_type=jax.ShapeDtypeStruct((ngrp, FIELDS, SUB, EMB), tables.dtype),
        mesh=mesh,
        scratch_types=[
            pltpu.VMEM((nw_,), jnp.int32),
            pltpu.VMEM((RING * G * WIN,), jnp.int32),
            pltpu.SemaphoreType.DMA,
            pltpu.SemaphoreType.DMA,
        ],
        compiler_params=cp,
    )
    def gather_kernel(x_hbm, i_hbm, o_hbm, raw_v, idx_ring, sem_a, sem_b):
        cid = lax.axis_index(("core", "subcore"))

        pltpu.sync_copy(i_hbm.at[pl.ds(cid * nw_, nw_)], raw_v)

        lane = lax.broadcasted_iota(jnp.int32, (16,), 0)
        bi16 = lane % SUB
        fh16 = lane // SUB

        def permute_step(s):
            slot = s % RING

            @pl.loop(0, G)
            def _(k):
                @pl.loop(0, FIELDS // 2)
                def _(j):
                    src16 = (s * G + k) * WIN + FIELDS * bi16 + 2 * j + fh16
                    vals = plsc.load_gather(raw_v, [src16])
                    idx_ring[pl.ds((slot * G + k) * WIN + j * 16, 16)] = vals

        permute_step(0)
        permute_step(1)

        def body(grid_idx, o_vmem):
            (i,) = grid_idx
            li = i - cid * nsteps_w
            slot = li % RING
            o_flat = o_vmem.reshape(G * WIN, EMB)
            ca = pltpu.make_async_copy(
                x_hbm.at[idx_ring.at[pl.ds(slot * G * WIN, WIN)]],
                o_flat.at[pl.ds(0, WIN), :],
                sem_a,
            )
            cb = pltpu.make_async_copy(
                x_hbm.at[idx_ring.at[pl.ds(slot * G * WIN + WIN, WIN)]],
                o_flat.at[pl.ds(WIN, WIN), :],
                sem_b,
            )
            ca.start()
            cb.start()

            @pl.when(li + 2 < nsteps_w)
            def _():
                permute_step(li + 2)

            ca.wait()
            cb.wait()

        pltpu.emit_pipeline(
            body,
            grid=(ngrp // G,),
            out_specs=[
                pl.BlockSpec((G, FIELDS, SUB, EMB), index_map=lambda i: (i, 0, 0, 0))
            ],
            core_axis_name=("core", "subcore"),
            dimension_semantics=(pltpu.PARALLEL,),
            _explicit_indices=True,
        )(o_hbm)

    out4 = gather_kernel(flat_tables, biased)
    return out4.transpose(0, 2, 1, 3).reshape(batch, FIELDS * EMB)
